# baseline (speedup 1.0000x reference)
import functools
import numpy as np
import jax
import jax.numpy as jnp
from jax import lax
from jax.experimental import pallas as pl
from jax.experimental.pallas import tpu as pltpu
from jax.experimental.pallas import tpu_sc as plsc

MID = 64
NH = 4
NP = 4
HD = 16
BN_EPS = 1e-3
NB_IMG = 4
H = W = 384
HF = WF = 96
N_PTS = 100000

BLK = 2000
NW = 32
CP = 32
NCHUNKS = N_PTS // CP
ROW_Y = W * NH
ROW_B = H * W * NH


def _a_body(pf_ref, pc_ref, wk_ref, bk_ref, wso_ref, bso_ref,
            wa_ref, ba_ref, meta_ref, base_ref, filt_ref):
    pf = pf_ref[...]
    q = pf @ wk_ref[...] + bk_ref[...]
    so = q @ wso_ref[...] + bso_ref[...]
    a = q @ wa_ref[...] + ba_ref[...]
    ap = [a[:, 4 * p:4 * p + 4] for p in range(NP)]
    m = jnp.maximum(jnp.maximum(ap[0], ap[1]), jnp.maximum(ap[2], ap[3]))
    ep = [jnp.exp(x - m) for x in ap]
    inv = 1.0 / (ep[0] + ep[1] + ep[2] + ep[3])
    aw = jnp.concatenate([e * inv for e in ep], axis=-1)

    pc = pc_ref[...]
    pxi = pc[:, 1:2].astype(jnp.int32)
    pyi = pc[:, 2:3].astype(jnp.int32)
    bi = pc[:, 0:1].astype(jnp.int32)
    filt = ((pxi >= 0) & (pxi < W) & (pyi >= 0) & (pyi < H))
    filt_ref[...] = filt.astype(jnp.float32)
    xf = pxi.astype(jnp.float32) + so[:, :16] - 0.5
    yf = pyi.astype(jnp.float32) + so[:, 16:] - 0.5
    hlane = lax.broadcasted_iota(jnp.int32, (BLK, 16), 1) % NH
    base_ref[...] = bi * ROW_B + hlane
    meta_ref[...] = jnp.concatenate([xf, yf, aw], axis=-1)


def _run_a(pf, pc, wkT, bk, wsoT, bso, waT, ba):
    nb = N_PTS // BLK
    return pl.pallas_call(
        _a_body,
        grid=(nb,),
        in_specs=[
            pl.BlockSpec((BLK, MID), lambda i: (i, 0)),
            pl.BlockSpec((BLK, 3), lambda i: (i, 0)),
            pl.BlockSpec((MID, MID), lambda i: (0, 0)),
            pl.BlockSpec((1, MID), lambda i: (0, 0)),
            pl.BlockSpec((MID, 2 * HD), lambda i: (0, 0)),
            pl.BlockSpec((1, 2 * HD), lambda i: (0, 0)),
            pl.BlockSpec((MID, HD), lambda i: (0, 0)),
            pl.BlockSpec((1, HD), lambda i: (0, 0)),
        ],
        out_specs=[
            pl.BlockSpec((BLK, 48), lambda i: (i, 0)),
            pl.BlockSpec((BLK, 16), lambda i: (i, 0)),
            pl.BlockSpec((BLK, 1), lambda i: (i, 0)),
        ],
        out_shape=[
            jax.ShapeDtypeStruct((N_PTS, 48), jnp.float32),
            jax.ShapeDtypeStruct((N_PTS, 16), jnp.int32),
            jax.ShapeDtypeStruct((N_PTS, 1), jnp.float32),
        ],
    )(pf, pc, wkT, bk, wsoT, bso, waT, ba)


def _b1_body(fpn_ref, wv_ref, bv_ref, out_ref):
    v = lax.dot_general(fpn_ref[0], wv_ref[...],
                        (((0,), (0,)), ((), ()))) + bv_ref[...]
    v = v.reshape(HF, WF, MID)
    am1 = jnp.concatenate([v[:1], v[:-1]], 0)
    ap1 = jnp.concatenate([v[1:], v[-1:]], 0)
    r0 = 0.375 * am1 + 0.625 * v
    r1 = 0.125 * am1 + 0.875 * v
    r2 = 0.875 * v + 0.125 * ap1
    r3 = 0.625 * v + 0.375 * ap1
    out_ref[0] = jnp.stack([r0, r1, r2, r3], axis=1)


def _run_b1(fpn_t, wvT, bv):
    return pl.pallas_call(
        _b1_body,
        grid=(NB_IMG,),
        in_specs=[
            pl.BlockSpec((1, MID, HF * WF), lambda b: (b, 0, 0)),
            pl.BlockSpec((MID, MID), lambda b: (0, 0)),
            pl.BlockSpec((1, MID), lambda b: (0, 0)),
        ],
        out_specs=pl.BlockSpec((1, HF, 4, WF, MID), lambda b: (b, 0, 0, 0, 0)),
        out_shape=jax.ShapeDtypeStruct((NB_IMG, HF, 4, WF, MID), jnp.float32),
    )(fpn_t, wvT, bv)


YB = 96


def _b2_body(y_ref, out_ref):
    v = y_ref[0].reshape(YB, WF, MID)
    am1 = jnp.concatenate([v[:, :1], v[:, :-1]], 1)
    ap1 = jnp.concatenate([v[:, 1:], v[:, -1:]], 1)
    r0 = 0.375 * am1 + 0.625 * v
    r1 = 0.125 * am1 + 0.875 * v
    r2 = 0.875 * v + 0.125 * ap1
    r3 = 0.625 * v + 0.375 * ap1
    out_ref[0] = jnp.stack([r0, r1, r2, r3], axis=2)


def _run_b2(yup):
    return pl.pallas_call(
        _b2_body,
        grid=(NB_IMG, H // YB),
        in_specs=[pl.BlockSpec((1, YB // 4, 4, WF, MID),
                               lambda b, i: (b, i, 0, 0, 0))],
        out_specs=pl.BlockSpec((1, YB, WF, 4, MID),
                               lambda b, i: (b, i, 0, 0, 0)),
        out_shape=jax.ShapeDtypeStruct((NB_IMG, H, WF, 4, MID), jnp.float32),
    )(yup)


def _bcast_lane(v, lane):
    idx = jnp.full((16, 1), lane, jnp.int32)
    dn = lax.GatherDimensionNumbers(
        offset_dims=(), collapsed_slice_dims=(0,), start_index_map=(0,))
    return lax.gather(v, idx, dn, (1,),
                      mode=lax.GatherScatterMode.PROMISE_IN_BOUNDS)


NITER = -(-NCHUNKS // NW)


def _sc_body(meta_hbm, base_hbm, table_hbm, out_hbm, meta_v, base_v, idx_v,
             w_v, rows_v, out_v, sems):
    wid = lax.axis_index("s") * 2 + lax.axis_index("c")

    def prep(c, buf):
        pltpu.sync_copy(meta_hbm.at[pl.ds(c * CP, CP), :], meta_v.at[buf])
        pltpu.sync_copy(base_hbm.at[pl.ds(c * CP, CP), :], base_v.at[buf])

        def point_idx(j, _):
            xfv = meta_v[buf, j, pl.ds(0, 16)]
            yfv = meta_v[buf, j, pl.ds(16, 16)]
            awv = meta_v[buf, j, pl.ds(32, 16)]
            bsv = base_v[buf, j, :]
            tx = xfv.astype(jnp.int32)
            x0 = jnp.where(tx.astype(jnp.float32) > xfv, tx - 1, tx)
            fx = xfv - x0.astype(jnp.float32)
            ty = yfv.astype(jnp.int32)
            y0 = jnp.where(ty.astype(jnp.float32) > yfv, ty - 1, ty)
            fy = yfv - y0.astype(jnp.float32)
            x1 = x0 + 1
            y1 = y0 + 1
            inx0 = (x0 >= 0) & (x0 < W)
            inx1 = (x1 >= 0) & (x1 < W)
            iny0 = (y0 >= 0) & (y0 < H)
            iny1 = (y1 >= 0) & (y1 < H)
            cx0 = jnp.minimum(jnp.maximum(x0, 0), W - 1) * NH
            cx1 = jnp.minimum(jnp.maximum(x1, 0), W - 1) * NH
            ry0 = jnp.minimum(jnp.maximum(y0, 0), H - 1) * ROW_Y
            ry1 = jnp.minimum(jnp.maximum(y1, 0), H - 1) * ROW_Y
            wx0 = (1.0 - fx) * awv
            wy0 = 1.0 - fy
            wx1 = fx * awv
            r = j >> 1
            s = (j & 1) * 64
            idx_v[buf, r, pl.ds(s, 16)] = bsv + ry0 + cx0
            idx_v[buf, r, pl.ds(s + 16, 16)] = bsv + ry0 + cx1
            idx_v[buf, r, pl.ds(s + 32, 16)] = bsv + ry1 + cx0
            idx_v[buf, r, pl.ds(s + 48, 16)] = bsv + ry1 + cx1
            w_v[buf, pl.ds(j * 64, 16)] = jnp.where(inx0 & iny0, wx0 * wy0,
                                                    0.0)
            w_v[buf, pl.ds(j * 64 + 16, 16)] = jnp.where(inx1 & iny0,
                                                         wx1 * wy0, 0.0)
            w_v[buf, pl.ds(j * 64 + 32, 16)] = jnp.where(inx0 & iny1,
                                                         wx0 * fy, 0.0)
            w_v[buf, pl.ds(j * 64 + 48, 16)] = jnp.where(inx1 & iny1,
                                                         wx1 * fy, 0.0)
            return 0

        lax.fori_loop(0, CP, point_idx, 0)
        for k in range(16):
            pltpu.async_copy(table_hbm.at[idx_v.at[buf].at[k]],
                             rows_v.at[buf].at[pl.ds(k * 128, 128)],
                             sems.at[buf])

    def drain_acc(c, buf):
        for k in range(16):
            pltpu.make_async_copy(
                table_hbm.at[pl.ds(0, 128)],
                rows_v.at[buf].at[pl.ds(k * 128, 128)],
                sems.at[buf]).wait()

        def point_acc(j, _):
            accs = []
            for h in range(NH):
                accs.append(jnp.zeros((16,), jnp.float32))
            for t in range(4):
                wv = w_v[buf, pl.ds(j * 64 + t * 16, 16)]
                for lane in range(16):
                    row = rows_v[buf, j * 64 + t * 16 + lane, :]
                    h = lane % NH
                    accs[h] = accs[h] + _bcast_lane(wv, lane) * row
            for h in range(NH):
                out_v[buf, j, pl.ds(h * HD, HD)] = accs[h]
            return 0

        lax.fori_loop(0, CP, point_acc, 0)
        pltpu.sync_copy(out_v.at[buf], out_hbm.at[pl.ds(c * CP, CP), :])

    prep(wid, 0)

    def sub(i, buf):
        c = i * NW + wid
        c_nxt = c + NW

        @pl.when(c_nxt < NCHUNKS)
        def _():
            prep(c_nxt, 1 - buf)

        @pl.when(c < NCHUNKS)
        def _():
            drain_acc(c, buf)

    def iter2(k, _):
        sub(2 * k, 0)
        sub(2 * k + 1, 1)
        return 0

    lax.fori_loop(0, NITER // 2, iter2, 0)


@functools.lru_cache(maxsize=1)
def _get_sc_kernel():
    return pl.kernel(
        _sc_body,
        out_type=jax.ShapeDtypeStruct((N_PTS, MID), jnp.float32),
        mesh=plsc.VectorSubcoreMesh(core_axis_name="c", subcore_axis_name="s"),
        scratch_types=[
            pltpu.VMEM((2, CP, 48), jnp.float32),
            pltpu.VMEM((2, CP, 16), jnp.int32),
            pltpu.VMEM((2, 16, 128), jnp.int32),
            pltpu.VMEM((2, CP * 64), jnp.float32),
            pltpu.VMEM((2, CP * 64, HD), jnp.float32),
            pltpu.VMEM((2, CP, MID), jnp.float32),
            pltpu.SemaphoreType.DMA((2,)),
        ],
        compiler_params=pltpu.CompilerParams(use_tc_tiling_on_sc=False),
    )


def _sc_sample(meta, base, table):
    return _get_sc_kernel()(meta, base, table)


def _c_body(attn_ref, pf_ref, filt_ref, wk_ref, bk_ref, wo_ref, bo_ref,
            wt_ref, bt_ref, wf1_ref, wf2_ref, bf_ref, out_ref):
    pf = pf_ref[...]
    q = pf @ wk_ref[...] + bk_ref[...]
    out = attn_ref[...] @ wo_ref[...] + bo_ref[...]
    sel = jnp.where(filt_ref[...] > 0.0, out, q)
    ppf = pf @ wt_ref[...] + bt_ref[...]
    h1 = jnp.maximum(ppf, 0.0)
    h2 = jnp.maximum(sel, 0.0)
    res = h1 @ wf1_ref[...] + h2 @ wf2_ref[...] + bf_ref[...]
    out_ref[...] = jnp.maximum(res, 0.0)


def _run_c(attn, pf, filt, wkT, bk, woT, bo, wtT, bt, wf1T, wf2T, bf):
    nb = N_PTS // BLK
    return pl.pallas_call(
        _c_body,
        grid=(nb,),
        in_specs=[
            pl.BlockSpec((BLK, MID), lambda i: (i, 0)),
            pl.BlockSpec((BLK, MID), lambda i: (i, 0)),
            pl.BlockSpec((BLK, 1), lambda i: (i, 0)),
            pl.BlockSpec((MID, MID), lambda i: (0, 0)),
            pl.BlockSpec((1, MID), lambda i: (0, 0)),
            pl.BlockSpec((MID, MID), lambda i: (0, 0)),
            pl.BlockSpec((1, MID), lambda i: (0, 0)),
            pl.BlockSpec((MID, MID), lambda i: (0, 0)),
            pl.BlockSpec((1, MID), lambda i: (0, 0)),
            pl.BlockSpec((MID, MID), lambda i: (0, 0)),
            pl.BlockSpec((MID, MID), lambda i: (0, 0)),
            pl.BlockSpec((1, MID), lambda i: (0, 0)),
        ],
        out_specs=pl.BlockSpec((BLK, MID), lambda i: (i, 0)),
        out_shape=jax.ShapeDtypeStruct((N_PTS, MID), jnp.float32),
    )(attn, pf, filt, wkT, bk, woT, bo, wtT, bt, wf1T, wf2T, bf)


def kernel(point_features, proj_coords, images, image_fpn_0, Wk, bk, gk,
           betak, Wt, bt, gt, betat, Wso, bso, Wa, ba, Wv, bv, Wo, bo, Wf, bf,
           gf, betaf):
    s = 1.0 / np.sqrt(1.0 + BN_EPS)
    wk = Wk * (s * gk)[:, None]
    bk2 = bk * s * gk + betak
    wt = Wt * (s * gt)[:, None]
    bt2 = bt * s * gt + betat
    wf = Wf * (s * gf)[:, None]
    bf2 = bf * s * gf + betaf
    wso_p = Wso.reshape(NH, NP, 2, MID).transpose(2, 1, 0, 3).reshape(32, MID)
    bso_p = bso.reshape(NH, NP, 2).transpose(2, 1, 0).reshape(32)
    wa_p = Wa.reshape(NH, NP, MID).transpose(1, 0, 2).reshape(16, MID)
    ba_p = ba.reshape(NH, NP).transpose(1, 0).reshape(16)

    pf = point_features
    meta, base, filt = _run_a(pf, proj_coords, wk.T, bk2[None], wso_p.T,
                              bso_p[None], wa_p.T, ba_p[None])

    fpn2 = image_fpn_0.reshape(NB_IMG, MID, HF * WF)
    yup = _run_b1(fpn2, Wv.T, bv[None])
    val = _run_b2(yup)
    table = val.reshape(NB_IMG * H * W * NH, HD)

    attn = _sc_sample(meta, base, table)

    return _run_c(attn, pf, filt, wk.T, bk2[None], Wo.T, bo[None], wt.T,
                  bt2[None], wf[:, :MID].T, wf[:, MID:].T, bf2[None])


# speedup vs baseline: 1.0068x; 1.0068x over previous
import functools
import numpy as np
import jax
import jax.numpy as jnp
from jax import lax
from jax.experimental import pallas as pl
from jax.experimental.pallas import tpu as pltpu
from jax.experimental.pallas import tpu_sc as plsc

MID = 64
NH = 4
NP = 4
HD = 16
BN_EPS = 1e-3
NB_IMG = 4
H = W = 384
HF = WF = 96
N_PTS = 100000

BLK = 2000
NW = 32
CP = 32
NCHUNKS = N_PTS // CP
ROW_Y = W * NH
ROW_B = H * W * NH


def _a_body(pf_ref, pc_ref, wk_ref, bk_ref, wso_ref, bso_ref,
            wa_ref, ba_ref, meta_ref, base_ref, filt_ref):
    pf = pf_ref[...]
    q = pf @ wk_ref[...] + bk_ref[...]
    so = q @ wso_ref[...] + bso_ref[...]
    a = q @ wa_ref[...] + ba_ref[...]
    ap = [a[:, 4 * p:4 * p + 4] for p in range(NP)]
    m = jnp.maximum(jnp.maximum(ap[0], ap[1]), jnp.maximum(ap[2], ap[3]))
    ep = [jnp.exp(x - m) for x in ap]
    inv = 1.0 / (ep[0] + ep[1] + ep[2] + ep[3])
    aw = jnp.concatenate([e * inv for e in ep], axis=-1)

    pc = pc_ref[...]
    pxi = pc[:, 1:2].astype(jnp.int32)
    pyi = pc[:, 2:3].astype(jnp.int32)
    bi = pc[:, 0:1].astype(jnp.int32)
    filt = ((pxi >= 0) & (pxi < W) & (pyi >= 0) & (pyi < H))
    filt_ref[...] = filt.astype(jnp.float32)
    xf = pxi.astype(jnp.float32) + so[:, :16] - 0.5
    yf = pyi.astype(jnp.float32) + so[:, 16:] - 0.5
    hlane = lax.broadcasted_iota(jnp.int32, (BLK, 16), 1) % NH
    base_ref[...] = bi * ROW_B + hlane
    meta_ref[...] = jnp.concatenate([xf, yf, aw], axis=-1)


def _run_a(pf, pc, wkT, bk, wsoT, bso, waT, ba):
    nb = N_PTS // BLK
    return pl.pallas_call(
        _a_body,
        grid=(nb,),
        in_specs=[
            pl.BlockSpec((BLK, MID), lambda i: (i, 0)),
            pl.BlockSpec((BLK, 3), lambda i: (i, 0)),
            pl.BlockSpec((MID, MID), lambda i: (0, 0)),
            pl.BlockSpec((1, MID), lambda i: (0, 0)),
            pl.BlockSpec((MID, 2 * HD), lambda i: (0, 0)),
            pl.BlockSpec((1, 2 * HD), lambda i: (0, 0)),
            pl.BlockSpec((MID, HD), lambda i: (0, 0)),
            pl.BlockSpec((1, HD), lambda i: (0, 0)),
        ],
        out_specs=[
            pl.BlockSpec((BLK, 48), lambda i: (i, 0)),
            pl.BlockSpec((BLK, 16), lambda i: (i, 0)),
            pl.BlockSpec((BLK, 1), lambda i: (i, 0)),
        ],
        out_shape=[
            jax.ShapeDtypeStruct((N_PTS, 48), jnp.float32),
            jax.ShapeDtypeStruct((N_PTS, 16), jnp.int32),
            jax.ShapeDtypeStruct((N_PTS, 1), jnp.float32),
        ],
    )(pf, pc, wkT, bk, wsoT, bso, waT, ba)


def _b1_body(fpn_ref, wv_ref, bv_ref, out_ref):
    v = fpn_ref[0].reshape(HF * WF, MID) @ wv_ref[...] + bv_ref[...]
    v = v.reshape(HF, WF, MID)
    am1 = jnp.concatenate([v[:1], v[:-1]], 0)
    ap1 = jnp.concatenate([v[1:], v[-1:]], 0)
    r0 = 0.375 * am1 + 0.625 * v
    r1 = 0.125 * am1 + 0.875 * v
    r2 = 0.875 * v + 0.125 * ap1
    r3 = 0.625 * v + 0.375 * ap1
    out_ref[0] = jnp.stack([r0, r1, r2, r3], axis=1)


def _run_b1(fpn_t, wvT, bv):
    return pl.pallas_call(
        _b1_body,
        grid=(NB_IMG,),
        in_specs=[
            pl.BlockSpec((1, HF, WF, MID), lambda b: (b, 0, 0, 0)),
            pl.BlockSpec((MID, MID), lambda b: (0, 0)),
            pl.BlockSpec((1, MID), lambda b: (0, 0)),
        ],
        out_specs=pl.BlockSpec((1, HF, 4, WF, MID), lambda b: (b, 0, 0, 0, 0)),
        out_shape=jax.ShapeDtypeStruct((NB_IMG, HF, 4, WF, MID), jnp.float32),
    )(fpn_t, wvT, bv)


YB = 48


def _b2_body(y_ref, out_ref):
    v = y_ref[0].reshape(YB, WF, MID)
    am1 = jnp.concatenate([v[:, :1], v[:, :-1]], 1)
    ap1 = jnp.concatenate([v[:, 1:], v[:, -1:]], 1)
    r0 = 0.375 * am1 + 0.625 * v
    r1 = 0.125 * am1 + 0.875 * v
    r2 = 0.875 * v + 0.125 * ap1
    r3 = 0.625 * v + 0.375 * ap1
    out_ref[0] = jnp.stack([r0, r1, r2, r3], axis=2)


def _run_b2(yup):
    return pl.pallas_call(
        _b2_body,
        grid=(NB_IMG, H // YB),
        in_specs=[pl.BlockSpec((1, YB // 4, 4, WF, MID),
                               lambda b, i: (b, i, 0, 0, 0))],
        out_specs=pl.BlockSpec((1, YB, WF, 4, MID),
                               lambda b, i: (b, i, 0, 0, 0)),
        out_shape=jax.ShapeDtypeStruct((NB_IMG, H, WF, 4, MID), jnp.float32),
    )(yup)


def _bcast_lane(v, lane):
    idx = jnp.full((16, 1), lane, jnp.int32)
    dn = lax.GatherDimensionNumbers(
        offset_dims=(), collapsed_slice_dims=(0,), start_index_map=(0,))
    return lax.gather(v, idx, dn, (1,),
                      mode=lax.GatherScatterMode.PROMISE_IN_BOUNDS)


NITER = -(-NCHUNKS // NW)


def _sc_body(meta_hbm, base_hbm, table_hbm, out_hbm, meta_v, base_v, idx_v,
             w_v, rows_v, out_v, sems):
    wid = lax.axis_index("s") * 2 + lax.axis_index("c")

    def prep(c, buf):
        pltpu.sync_copy(meta_hbm.at[pl.ds(c * CP, CP), :], meta_v.at[buf])
        pltpu.sync_copy(base_hbm.at[pl.ds(c * CP, CP), :], base_v.at[buf])

        def point_idx(j, _):
            xfv = meta_v[buf, j, pl.ds(0, 16)]
            yfv = meta_v[buf, j, pl.ds(16, 16)]
            awv = meta_v[buf, j, pl.ds(32, 16)]
            bsv = base_v[buf, j, :]
            tx = xfv.astype(jnp.int32)
            x0 = jnp.where(tx.astype(jnp.float32) > xfv, tx - 1, tx)
            fx = xfv - x0.astype(jnp.float32)
            ty = yfv.astype(jnp.int32)
            y0 = jnp.where(ty.astype(jnp.float32) > yfv, ty - 1, ty)
            fy = yfv - y0.astype(jnp.float32)
            x1 = x0 + 1
            y1 = y0 + 1
            inx0 = (x0 >= 0) & (x0 < W)
            inx1 = (x1 >= 0) & (x1 < W)
            iny0 = (y0 >= 0) & (y0 < H)
            iny1 = (y1 >= 0) & (y1 < H)
            cx0 = jnp.minimum(jnp.maximum(x0, 0), W - 1) * NH
            cx1 = jnp.minimum(jnp.maximum(x1, 0), W - 1) * NH
            ry0 = jnp.minimum(jnp.maximum(y0, 0), H - 1) * ROW_Y
            ry1 = jnp.minimum(jnp.maximum(y1, 0), H - 1) * ROW_Y
            wx0 = (1.0 - fx) * awv
            wy0 = 1.0 - fy
            wx1 = fx * awv
            r = j >> 1
            s = (j & 1) * 64
            idx_v[buf, r, pl.ds(s, 16)] = bsv + ry0 + cx0
            idx_v[buf, r, pl.ds(s + 16, 16)] = bsv + ry0 + cx1
            idx_v[buf, r, pl.ds(s + 32, 16)] = bsv + ry1 + cx0
            idx_v[buf, r, pl.ds(s + 48, 16)] = bsv + ry1 + cx1
            w_v[buf, pl.ds(j * 64, 16)] = jnp.where(inx0 & iny0, wx0 * wy0,
                                                    0.0)
            w_v[buf, pl.ds(j * 64 + 16, 16)] = jnp.where(inx1 & iny0,
                                                         wx1 * wy0, 0.0)
            w_v[buf, pl.ds(j * 64 + 32, 16)] = jnp.where(inx0 & iny1,
                                                         wx0 * fy, 0.0)
            w_v[buf, pl.ds(j * 64 + 48, 16)] = jnp.where(inx1 & iny1,
                                                         wx1 * fy, 0.0)
            return 0

        lax.fori_loop(0, CP, point_idx, 0)
        for k in range(16):
            pltpu.async_copy(table_hbm.at[idx_v.at[buf].at[k]],
                             rows_v.at[buf].at[pl.ds(k * 128, 128)],
                             sems.at[buf])

    def drain_acc(c, buf):
        for k in range(16):
            pltpu.make_async_copy(
                table_hbm.at[pl.ds(0, 128)],
                rows_v.at[buf].at[pl.ds(k * 128, 128)],
                sems.at[buf]).wait()

        def point_acc(j, _):
            accs = []
            for h in range(NH):
                accs.append(jnp.zeros((16,), jnp.float32))
            for t in range(4):
                wv = w_v[buf, pl.ds(j * 64 + t * 16, 16)]
                for lane in range(16):
                    row = rows_v[buf, j * 64 + t * 16 + lane, :]
                    h = lane % NH
                    accs[h] = accs[h] + _bcast_lane(wv, lane) * row
            for h in range(NH):
                out_v[buf, j, pl.ds(h * HD, HD)] = accs[h]
            return 0

        lax.fori_loop(0, CP, point_acc, 0)
        pltpu.sync_copy(out_v.at[buf], out_hbm.at[pl.ds(c * CP, CP), :])

    prep(wid, 0)

    def sub(i, buf):
        c = i * NW + wid
        c_nxt = c + NW

        @pl.when(c_nxt < NCHUNKS)
        def _():
            prep(c_nxt, 1 - buf)

        @pl.when(c < NCHUNKS)
        def _():
            drain_acc(c, buf)

    def iter2(k, _):
        sub(2 * k, 0)
        sub(2 * k + 1, 1)
        return 0

    lax.fori_loop(0, NITER // 2, iter2, 0)


@functools.lru_cache(maxsize=1)
def _get_sc_kernel():
    return pl.kernel(
        _sc_body,
        out_type=jax.ShapeDtypeStruct((N_PTS, MID), jnp.float32),
        mesh=plsc.VectorSubcoreMesh(core_axis_name="c", subcore_axis_name="s"),
        scratch_types=[
            pltpu.VMEM((2, CP, 48), jnp.float32),
            pltpu.VMEM((2, CP, 16), jnp.int32),
            pltpu.VMEM((2, 16, 128), jnp.int32),
            pltpu.VMEM((2, CP * 64), jnp.float32),
            pltpu.VMEM((2, CP * 64, HD), jnp.float32),
            pltpu.VMEM((2, CP, MID), jnp.float32),
            pltpu.SemaphoreType.DMA((2,)),
        ],
        compiler_params=pltpu.CompilerParams(use_tc_tiling_on_sc=False),
    )


def _sc_sample(meta, base, table):
    return _get_sc_kernel()(meta, base, table)


def _c_body(attn_ref, pf_ref, filt_ref, wk_ref, bk_ref, wo_ref, bo_ref,
            wt_ref, bt_ref, wf1_ref, wf2_ref, bf_ref, out_ref):
    pf = pf_ref[...]
    q = pf @ wk_ref[...] + bk_ref[...]
    out = attn_ref[...] @ wo_ref[...] + bo_ref[...]
    sel = jnp.where(filt_ref[...] > 0.0, out, q)
    ppf = pf @ wt_ref[...] + bt_ref[...]
    h1 = jnp.maximum(ppf, 0.0)
    h2 = jnp.maximum(sel, 0.0)
    res = h1 @ wf1_ref[...] + h2 @ wf2_ref[...] + bf_ref[...]
    out_ref[...] = jnp.maximum(res, 0.0)


def _run_c(attn, pf, filt, wkT, bk, woT, bo, wtT, bt, wf1T, wf2T, bf):
    nb = N_PTS // BLK
    return pl.pallas_call(
        _c_body,
        grid=(nb,),
        in_specs=[
            pl.BlockSpec((BLK, MID), lambda i: (i, 0)),
            pl.BlockSpec((BLK, MID), lambda i: (i, 0)),
            pl.BlockSpec((BLK, 1), lambda i: (i, 0)),
            pl.BlockSpec((MID, MID), lambda i: (0, 0)),
            pl.BlockSpec((1, MID), lambda i: (0, 0)),
            pl.BlockSpec((MID, MID), lambda i: (0, 0)),
            pl.BlockSpec((1, MID), lambda i: (0, 0)),
            pl.BlockSpec((MID, MID), lambda i: (0, 0)),
            pl.BlockSpec((1, MID), lambda i: (0, 0)),
            pl.BlockSpec((MID, MID), lambda i: (0, 0)),
            pl.BlockSpec((MID, MID), lambda i: (0, 0)),
            pl.BlockSpec((1, MID), lambda i: (0, 0)),
        ],
        out_specs=pl.BlockSpec((BLK, MID), lambda i: (i, 0)),
        out_shape=jax.ShapeDtypeStruct((N_PTS, MID), jnp.float32),
    )(attn, pf, filt, wkT, bk, woT, bo, wtT, bt, wf1T, wf2T, bf)


def kernel(point_features, proj_coords, images, image_fpn_0, Wk, bk, gk,
           betak, Wt, bt, gt, betat, Wso, bso, Wa, ba, Wv, bv, Wo, bo, Wf, bf,
           gf, betaf):
    s = 1.0 / np.sqrt(1.0 + BN_EPS)
    wk = Wk * (s * gk)[:, None]
    bk2 = bk * s * gk + betak
    wt = Wt * (s * gt)[:, None]
    bt2 = bt * s * gt + betat
    wf = Wf * (s * gf)[:, None]
    bf2 = bf * s * gf + betaf
    wso_p = Wso.reshape(NH, NP, 2, MID).transpose(2, 1, 0, 3).reshape(32, MID)
    bso_p = bso.reshape(NH, NP, 2).transpose(2, 1, 0).reshape(32)
    wa_p = Wa.reshape(NH, NP, MID).transpose(1, 0, 2).reshape(16, MID)
    ba_p = ba.reshape(NH, NP).transpose(1, 0).reshape(16)

    pf = point_features
    meta, base, filt = _run_a(pf, proj_coords, wk.T, bk2[None], wso_p.T,
                              bso_p[None], wa_p.T, ba_p[None])

    fpn_t = image_fpn_0.transpose(0, 2, 3, 1)
    yup = _run_b1(fpn_t, Wv.T, bv[None])
    val = _run_b2(yup)
    table = val.reshape(NB_IMG * H * W * NH, HD)

    attn = _sc_sample(meta, base, table)

    return _run_c(attn, pf, filt, wk.T, bk2[None], Wo.T, bo[None], wt.T,
                  bt2[None], wf[:, :MID].T, wf[:, MID:].T, bf2[None])
